# baseline (speedup 1.0000x reference)
import functools
import math

import jax
import jax.numpy as jnp
from jax import lax
from jax.experimental import pallas as pl
from jax.experimental.pallas import tpu as pltpu

H = 16
K = 8


def _fold_body(lat_ref, wq_ref, bq_ref, wv_ref, bv_ref, wc_ref, bc_ref, *,
               num_l):
    lat = lat_ref[0]
    wl = lax.dot_general(lat, wq_ref[...], (((1,), (0,)), ((), ())),
                         preferred_element_type=jnp.float32)
    wc_ref[0, :num_l, :] = wl
    wc_ref[0, num_l:, :] = wv_ref[...]
    bl = lax.dot_general(bq_ref[0], lat, (((1,), (1,)), ((), ())),
                         preferred_element_type=jnp.float32)
    bc_ref[0, :, :num_l] = bl
    bc_ref[0, :, num_l:] = bv_ref[0]


def _msla_body(x_ref, wc_ref, bc_ref, lat_ref, wo_ref, bo_ref, o_ref, *,
               num_k, num_l, scale):
    hi = pl.program_id(2)
    dn = (((1,), (1,)), ((), ()))

    x = x_ref[0]
    lv = lax.dot_general(x, wc_ref[0], dn,
                         preferred_element_type=jnp.float32) + bc_ref[0]
    logits = lv[:, :num_l] * scale
    v = lv[:, num_l:]

    work = logits
    mask = jnp.zeros(logits.shape, jnp.bool_)
    mx = None
    z = None
    for k in range(num_k):
        m = jnp.max(work, axis=1, keepdims=True)
        if k == 0:
            mx = m
            z = jnp.ones_like(m)
        else:
            z = z + jnp.exp(m - mx)
        sel = work == m
        mask = jnp.logical_or(mask, sel)
        work = jnp.where(sel, -jnp.inf, work)

    p = jnp.where(mask, jnp.exp(logits - mx), 0.0) / z

    weighted = lax.dot_general(p, lat_ref[0], (((1,), (0,)), ((), ())),
                               preferred_element_type=jnp.float32)
    head = weighted + v
    contrib = lax.dot_general(head, wo_ref[...], dn,
                              preferred_element_type=jnp.float32)

    @pl.when(hi == 0)
    def _():
        o_ref[0] = contrib + bo_ref[...]

    @pl.when(hi != 0)
    def _():
        o_ref[0] += contrib


def kernel(hidden_states, Wq, bq, Wk, bk, Wv, bv, Wo, bo, latent_keys):
    del Wk, bk
    b, t, d = hidden_states.shape
    hd = d // H
    l = latent_keys.shape[0]
    tb = 512
    scale = 1.0 / math.sqrt(hd)

    bq_r = bq.reshape(H, 1, hd)
    bv_r = bv.reshape(H, 1, hd)
    lat_r = latent_keys.reshape(l, H, hd).transpose(1, 0, 2)
    bo_r = bo.reshape(1, d)

    wc, bc = pl.pallas_call(
        functools.partial(_fold_body, num_l=l),
        grid=(H,),
        in_specs=[
            pl.BlockSpec((1, l, hd), lambda hi: (hi, 0, 0)),
            pl.BlockSpec((hd, d), lambda hi: (hi, 0)),
            pl.BlockSpec((1, 1, hd), lambda hi: (hi, 0, 0)),
            pl.BlockSpec((hd, d), lambda hi: (hi, 0)),
            pl.BlockSpec((1, 1, hd), lambda hi: (hi, 0, 0)),
        ],
        out_specs=[
            pl.BlockSpec((1, l + hd, d), lambda hi: (hi, 0, 0)),
            pl.BlockSpec((1, 1, l + hd), lambda hi: (hi, 0, 0)),
        ],
        out_shape=[
            jax.ShapeDtypeStruct((H, l + hd, d), jnp.float32),
            jax.ShapeDtypeStruct((H, 1, l + hd), jnp.float32),
        ],
    )(lat_r, Wq, bq_r, Wv, bv_r)

    grid = (b, t // tb, H)
    body = functools.partial(_msla_body, num_k=K, num_l=l, scale=scale)
    out = pl.pallas_call(
        body,
        grid=grid,
        in_specs=[
            pl.BlockSpec((1, tb, d), lambda bi, ti, hi: (bi, ti, 0)),
            pl.BlockSpec((1, l + hd, d), lambda bi, ti, hi: (hi, 0, 0)),
            pl.BlockSpec((1, 1, l + hd), lambda bi, ti, hi: (hi, 0, 0)),
            pl.BlockSpec((1, l, hd), lambda bi, ti, hi: (hi, 0, 0)),
            pl.BlockSpec((d, hd), lambda bi, ti, hi: (0, hi)),
            pl.BlockSpec((1, d), lambda bi, ti, hi: (0, 0)),
        ],
        out_specs=pl.BlockSpec((1, tb, d), lambda bi, ti, hi: (bi, ti, 0)),
        out_shape=jax.ShapeDtypeStruct((b, t, d), jnp.float32),
        compiler_params=pltpu.CompilerParams(
            dimension_semantics=("parallel", "parallel", "arbitrary"),
        ),
    )(hidden_states, wc, bc, lat_r, Wo, bo_r)
    return out


# speedup vs baseline: 1.4593x; 1.4593x over previous
import functools
import math

import jax
import jax.numpy as jnp
from jax import lax
from jax.experimental import pallas as pl
from jax.experimental.pallas import tpu as pltpu

H = 16
K = 8


def _msla_body(x_ref, wq_ref, bq_ref, wv_ref, bv_ref, lat_ref, wo_ref, bo_ref,
               o_ref, acc_s, *, num_k, num_h, scale):
    hi = pl.program_id(2)
    dn = (((1,), (1,)), ((), ()))
    hd = wq_ref.shape[0]

    x = x_ref[0]
    q = lax.dot_general(x, wq_ref[...], dn,
                        preferred_element_type=jnp.float32) + bq_ref[0]
    v = lax.dot_general(x, wv_ref[...], dn,
                        preferred_element_type=jnp.float32) + bv_ref[0]
    lat = lat_ref[0]
    logits = lax.dot_general(q, lat, dn,
                             preferred_element_type=jnp.float32) * scale

    work = logits
    mask = jnp.zeros(logits.shape, jnp.bool_)
    mx = None
    z = None
    for k in range(num_k):
        m = jnp.max(work, axis=1, keepdims=True)
        if k == 0:
            mx = m
            z = jnp.ones_like(m)
        else:
            z = z + jnp.exp(m - mx)
        sel = work == m
        mask = jnp.logical_or(mask, sel)
        work = jnp.where(sel, -jnp.inf, work)

    p = jnp.where(mask, jnp.exp(logits - mx), 0.0) / z

    weighted = lax.dot_general(p, lat, (((1,), (0,)), ((), ())),
                               preferred_element_type=jnp.float32)
    acc_s[:, pl.ds(hi * hd, hd)] = weighted + v

    @pl.when(hi == num_h - 1)
    def _():
        attn = acc_s[...]
        o_ref[0] = lax.dot_general(attn, wo_ref[...], dn,
                                   preferred_element_type=jnp.float32
                                   ) + bo_ref[...]


def kernel(hidden_states, Wq, bq, Wk, bk, Wv, bv, Wo, bo, latent_keys):
    del Wk, bk
    b, t, d = hidden_states.shape
    hd = d // H
    l = latent_keys.shape[0]
    tb = 512
    scale = 1.0 / math.sqrt(hd)

    bq_r = bq.reshape(H, 1, hd)
    bv_r = bv.reshape(H, 1, hd)
    lat_r = latent_keys.reshape(l, H, hd).transpose(1, 0, 2)
    bo_r = bo.reshape(1, d)

    grid = (b, t // tb, H)
    body = functools.partial(_msla_body, num_k=K, num_h=H, scale=scale)
    out = pl.pallas_call(
        body,
        grid=grid,
        in_specs=[
            pl.BlockSpec((1, tb, d), lambda bi, ti, hi: (bi, ti, 0)),
            pl.BlockSpec((hd, d), lambda bi, ti, hi: (hi, 0)),
            pl.BlockSpec((1, 1, hd), lambda bi, ti, hi: (hi, 0, 0)),
            pl.BlockSpec((hd, d), lambda bi, ti, hi: (hi, 0)),
            pl.BlockSpec((1, 1, hd), lambda bi, ti, hi: (hi, 0, 0)),
            pl.BlockSpec((1, l, hd), lambda bi, ti, hi: (hi, 0, 0)),
            pl.BlockSpec((d, d), lambda bi, ti, hi: (0, 0)),
            pl.BlockSpec((1, d), lambda bi, ti, hi: (0, 0)),
        ],
        out_specs=pl.BlockSpec((1, tb, d), lambda bi, ti, hi: (bi, ti, 0)),
        out_shape=jax.ShapeDtypeStruct((b, t, d), jnp.float32),
        scratch_shapes=[
            pltpu.MemorySpace.VMEM((tb, d), jnp.float32),
        ],
        compiler_params=pltpu.CompilerParams(
            dimension_semantics=("parallel", "parallel", "arbitrary"),
        ),
    )(hidden_states, Wq, bq_r, Wv, bv_r, lat_r, Wo, bo_r)
    return out


# speedup vs baseline: 1.5938x; 1.0921x over previous
import functools
import math

import jax
import jax.numpy as jnp
from jax import lax
from jax.experimental import pallas as pl
from jax.experimental.pallas import tpu as pltpu

H = 16
K = 8


def _msla_body(x_ref, wq_ref, bq_ref, wv_ref, bv_ref, lat_ref, wo_ref, bo_ref,
               o_ref, acc_s, *, num_k, num_h, scale):
    hi = pl.program_id(2)
    dn = (((1,), (1,)), ((), ()))
    hd = wq_ref.shape[0]

    x = x_ref[0]
    q = lax.dot_general(x, wq_ref[...], dn,
                        preferred_element_type=jnp.float32) + bq_ref[0]
    v = lax.dot_general(x, wv_ref[...], dn,
                        preferred_element_type=jnp.float32) + bv_ref[0]
    lat = lat_ref[0]
    logits = lax.dot_general(q, lat, dn,
                             preferred_element_type=jnp.float32) * scale

    work = logits
    mask = jnp.zeros(logits.shape, jnp.bool_)
    mx = None
    z = None
    for k in range(num_k):
        m = jnp.max(work, axis=1, keepdims=True)
        if k == 0:
            mx = m
            z = jnp.ones_like(m)
        else:
            z = z + jnp.exp(m - mx)
        sel = work == m
        mask = jnp.logical_or(mask, sel)
        work = jnp.where(sel, -jnp.inf, work)

    p = jnp.where(mask, jnp.exp(logits - mx), 0.0) / z

    weighted = lax.dot_general(p, lat, (((1,), (0,)), ((), ())),
                               preferred_element_type=jnp.float32)
    acc_s[:, pl.ds(hi * hd, hd)] = weighted + v

    @pl.when(hi == num_h - 1)
    def _():
        attn = acc_s[...]
        o_ref[0] = lax.dot_general(attn, wo_ref[...], dn,
                                   preferred_element_type=jnp.float32
                                   ) + bo_ref[...]


def kernel(hidden_states, Wq, bq, Wk, bk, Wv, bv, Wo, bo, latent_keys):
    del Wk, bk
    b, t, d = hidden_states.shape
    hd = d // H
    l = latent_keys.shape[0]
    tb = 1024
    scale = 1.0 / math.sqrt(hd)

    bq_r = bq.reshape(H, 1, hd)
    bv_r = bv.reshape(H, 1, hd)
    lat_r = latent_keys.reshape(l, H, hd).transpose(1, 0, 2)
    bo_r = bo.reshape(1, d)

    grid = (b, t // tb, H)
    body = functools.partial(_msla_body, num_k=K, num_h=H, scale=scale)
    out = pl.pallas_call(
        body,
        grid=grid,
        in_specs=[
            pl.BlockSpec((1, tb, d), lambda bi, ti, hi: (bi, ti, 0)),
            pl.BlockSpec((hd, d), lambda bi, ti, hi: (hi, 0)),
            pl.BlockSpec((1, 1, hd), lambda bi, ti, hi: (hi, 0, 0)),
            pl.BlockSpec((hd, d), lambda bi, ti, hi: (hi, 0)),
            pl.BlockSpec((1, 1, hd), lambda bi, ti, hi: (hi, 0, 0)),
            pl.BlockSpec((1, l, hd), lambda bi, ti, hi: (hi, 0, 0)),
            pl.BlockSpec((d, d), lambda bi, ti, hi: (0, 0)),
            pl.BlockSpec((1, d), lambda bi, ti, hi: (0, 0)),
        ],
        out_specs=pl.BlockSpec((1, tb, d), lambda bi, ti, hi: (bi, ti, 0),
                               pipeline_mode=pl.Buffered(buffer_count=1)),
        out_shape=jax.ShapeDtypeStruct((b, t, d), jnp.float32),
        scratch_shapes=[
            pltpu.MemorySpace.VMEM((tb, d), jnp.float32),
        ],
        compiler_params=pltpu.CompilerParams(
            dimension_semantics=("parallel", "parallel", "arbitrary"),
        ),
    )(hidden_states, Wq, bq_r, Wv, bv_r, lat_r, Wo, bo_r)
    return out
